# Initial kernel scaffold; baseline (speedup 1.0000x reference)
import functools

import jax
import jax.numpy as jnp
from jax.experimental import pallas as pl
from jax.experimental.pallas import tpu as pltpu

D_MODEL = 768
NUM_TILES = 8
D_FF = 3072
TILE_FF = D_FF // NUM_TILES
TILE_OUT = D_MODEL // NUM_TILES

BLK = 512


def _sig_kernel(up_ref, sig_ref):
    w = up_ref[...]
    s = w.reshape(NUM_TILES, TILE_FF, D_MODEL).sum(axis=1)
    sig_ref[...] = jnp.sign(s)


def _ffn_kernel(x_ref, sig_ref, up_ref, diag_ref, upb_ref, downb_ref,
                out_ref, gate_ref):
    x = x_ref[...]
    scores = jax.lax.dot_general(
        x, sig_ref[...], (((1,), (1,)), ((), ())),
        preferred_element_type=jnp.float32)
    m = jnp.max(scores, axis=-1, keepdims=True)
    eq = (scores == m)
    first = jnp.cumsum(eq.astype(jnp.int32), axis=-1) == 1
    gate = (eq & first).astype(jnp.float32)
    gate_ref[...] = gate

    h = jax.lax.dot_general(
        x, up_ref[...], (((1,), (1,)), ((), ())),
        preferred_element_type=jnp.float32)
    h = jnp.maximum(h + upb_ref[...], 0.0)

    parts = []
    for t in range(NUM_TILES):
        g_t = gate[:, t:t + 1]
        h_t = h[:, t * TILE_FF:(t + 1) * TILE_FF] * g_t
        o_t = jax.lax.dot_general(
            h_t, diag_ref[t], (((1,), (1,)), ((), ())),
            preferred_element_type=jnp.float32)
        o_t = (o_t + downb_ref[:, t * TILE_OUT:(t + 1) * TILE_OUT]) * g_t
        parts.append(o_t)
    out_ref[...] = jnp.concatenate(parts, axis=1)


def kernel(x, up_W, up_b, down_W, down_b):
    orig_shape = x.shape
    n = orig_shape[0] * orig_shape[1]
    xf = x.reshape(n, D_MODEL)

    sig = pl.pallas_call(
        _sig_kernel,
        out_shape=jax.ShapeDtypeStruct((NUM_TILES, D_MODEL), jnp.float32),
    )(up_W)

    diag = jnp.stack([
        jax.lax.slice(down_W, (t * TILE_OUT, t * TILE_FF),
                      ((t + 1) * TILE_OUT, (t + 1) * TILE_FF))
        for t in range(NUM_TILES)
    ])

    grid = (n // BLK,)
    out, gate = pl.pallas_call(
        _ffn_kernel,
        grid=grid,
        in_specs=[
            pl.BlockSpec((BLK, D_MODEL), lambda i: (i, 0)),
            pl.BlockSpec((NUM_TILES, D_MODEL), lambda i: (0, 0)),
            pl.BlockSpec((D_FF, D_MODEL), lambda i: (0, 0)),
            pl.BlockSpec((NUM_TILES, TILE_OUT, TILE_FF), lambda i: (0, 0, 0)),
            pl.BlockSpec((1, D_FF), lambda i: (0, 0)),
            pl.BlockSpec((1, D_MODEL), lambda i: (0, 0)),
        ],
        out_specs=[
            pl.BlockSpec((BLK, D_MODEL), lambda i: (i, 0)),
            pl.BlockSpec((BLK, NUM_TILES), lambda i: (i, 0)),
        ],
        out_shape=[
            jax.ShapeDtypeStruct((n, D_MODEL), jnp.float32),
            jax.ShapeDtypeStruct((n, NUM_TILES), jnp.float32),
        ],
        compiler_params=pltpu.CompilerParams(
            dimension_semantics=("arbitrary",),
        ),
    )(xf, sig, up_W, diag, up_b.reshape(1, D_FF), down_b.reshape(1, D_MODEL))

    return (out.reshape(orig_shape[0], orig_shape[1], D_MODEL),
            gate.reshape(orig_shape[0], orig_shape[1], NUM_TILES))


# speedup vs baseline: 3.2082x; 3.2082x over previous
import functools

import jax
import jax.numpy as jnp
from jax.experimental import pallas as pl
from jax.experimental.pallas import tpu as pltpu

D_MODEL = 768
NUM_TILES = 8
D_FF = 3072
TILE_FF = D_FF // NUM_TILES
TILE_OUT = D_MODEL // NUM_TILES

BLK = 512


def _sig_kernel(up_ref, sig_ref):
    w = up_ref[...]
    s = w.reshape(NUM_TILES, TILE_FF, D_MODEL).sum(axis=1)
    sig_ref[...] = jnp.sign(s)


def _ffn_kernel(x_ref, sig_ref, up_ref, diag_ref, upb_ref, downb_ref,
                out_ref, gate_ref):
    x = x_ref[...]
    scores = jax.lax.dot_general(
        x, sig_ref[...], (((1,), (1,)), ((), ())),
        preferred_element_type=jnp.float32)
    m = jnp.max(scores, axis=-1, keepdims=True)
    eq = (scores == m)
    idx = jax.lax.broadcasted_iota(jnp.int32, scores.shape, 1)
    winner = jnp.min(jnp.where(eq, idx, NUM_TILES), axis=-1, keepdims=True)
    gate = (idx == winner).astype(jnp.float32)
    gate_ref[...] = gate

    h = jax.lax.dot_general(
        x, up_ref[...], (((1,), (1,)), ((), ())),
        preferred_element_type=jnp.float32)
    h = jnp.maximum(h + upb_ref[...], 0.0)

    parts = []
    for t in range(NUM_TILES):
        g_t = gate[:, t:t + 1]
        h_t = h[:, t * TILE_FF:(t + 1) * TILE_FF] * g_t
        o_t = jax.lax.dot_general(
            h_t, diag_ref[t], (((1,), (1,)), ((), ())),
            preferred_element_type=jnp.float32)
        o_t = (o_t + downb_ref[:, t * TILE_OUT:(t + 1) * TILE_OUT]) * g_t
        parts.append(o_t)
    out_ref[...] = jnp.concatenate(parts, axis=1)


def kernel(x, up_W, up_b, down_W, down_b):
    orig_shape = x.shape
    n = orig_shape[0] * orig_shape[1]
    xf = x.reshape(n, D_MODEL)

    sig = pl.pallas_call(
        _sig_kernel,
        out_shape=jax.ShapeDtypeStruct((NUM_TILES, D_MODEL), jnp.float32),
    )(up_W)

    diag = jnp.stack([
        jax.lax.slice(down_W, (t * TILE_OUT, t * TILE_FF),
                      ((t + 1) * TILE_OUT, (t + 1) * TILE_FF))
        for t in range(NUM_TILES)
    ])

    grid = (n // BLK,)
    out, gate = pl.pallas_call(
        _ffn_kernel,
        grid=grid,
        in_specs=[
            pl.BlockSpec((BLK, D_MODEL), lambda i: (i, 0)),
            pl.BlockSpec((NUM_TILES, D_MODEL), lambda i: (0, 0)),
            pl.BlockSpec((D_FF, D_MODEL), lambda i: (0, 0)),
            pl.BlockSpec((NUM_TILES, TILE_OUT, TILE_FF), lambda i: (0, 0, 0)),
            pl.BlockSpec((1, D_FF), lambda i: (0, 0)),
            pl.BlockSpec((1, D_MODEL), lambda i: (0, 0)),
        ],
        out_specs=[
            pl.BlockSpec((BLK, D_MODEL), lambda i: (i, 0)),
            pl.BlockSpec((BLK, NUM_TILES), lambda i: (i, 0)),
        ],
        out_shape=[
            jax.ShapeDtypeStruct((n, D_MODEL), jnp.float32),
            jax.ShapeDtypeStruct((n, NUM_TILES), jnp.float32),
        ],
        compiler_params=pltpu.CompilerParams(
            dimension_semantics=("arbitrary",),
        ),
    )(xf, sig, up_W, diag, up_b.reshape(1, D_FF), down_b.reshape(1, D_MODEL))

    return (out.reshape(orig_shape[0], orig_shape[1], D_MODEL),
            gate.reshape(orig_shape[0], orig_shape[1], NUM_TILES))
